# initial kernel scaffold (unmeasured)
import math

import jax
import jax.numpy as jnp
from jax import lax
from jax.experimental import pallas as pl
from jax.experimental.pallas import tpu as pltpu

N_DEV = 4
HQ, DH = 4, 64


def kernel(x, Wq, Wk, Wv, Wo):
    B, S_LOC, D = x.shape
    DQ = Wq.shape[1]
    D_OUT = Wo.shape[1]
    S_GLB = N_DEV * S_LOC

    def body(x_ref, wq_ref, wk_ref, wv_ref, wo_ref, out_ref,
             kall, vall, ksend, krecv, vsend, vrecv):
        my = lax.axis_index("i")
        right = lax.rem(my + 1, N_DEV)
        left = lax.rem(my + N_DEV - 1, N_DEV)

        barrier = pltpu.get_barrier_semaphore()
        for nbr in (left, right):
            pl.semaphore_signal(barrier, inc=1, device_id=(nbr,),
                                device_id_type=pl.DeviceIdType.MESH)
        pl.semaphore_wait(barrier, 2)

        row = lax.broadcasted_iota(jnp.float32, (S_LOC, DQ), 0)
        col = lax.broadcasted_iota(jnp.int32, (S_LOC, DQ), 1)
        dd = col % DH
        dpair = ((dd // 2) * 2).astype(jnp.float32)
        freq = jnp.exp(dpair * (-math.log(10000.0) / DH))
        pos = row + (my * S_LOC).astype(jnp.float32)
        ang = pos * freq
        cosv = jnp.cos(ang)
        sinv = jnp.sin(ang)

        r_i = lax.broadcasted_iota(jnp.int32, (DQ, DQ), 0)
        c_i = lax.broadcasted_iota(jnp.int32, (DQ, DQ), 1)
        M = jnp.where((r_i == c_i + 1) & (c_i % 2 == 0), -1.0,
                      jnp.where((r_i + 1 == c_i) & (c_i % 2 == 1), 1.0,
                                0.0)).astype(jnp.bfloat16)

        def rope(t):
            tr = jnp.dot(t.astype(jnp.bfloat16), M,
                         preferred_element_type=jnp.float32)
            return (t * cosv + tr * sinv).astype(jnp.bfloat16)

        wq = wq_ref[...].astype(jnp.bfloat16)
        wk = wk_ref[...].astype(jnp.bfloat16)
        wv = wv_ref[...].astype(jnp.bfloat16)
        wo = wo_ref[...].astype(jnp.bfloat16)

        for b in range(B):
            xb = x_ref[b].astype(jnp.bfloat16)
            kb = jnp.dot(xb, wk, preferred_element_type=jnp.float32)
            vb = jnp.dot(xb, wv, preferred_element_type=jnp.float32)
            kall[0, b] = rope(kb)
            vall[0, b] = vb.astype(jnp.bfloat16)

        for h in range(N_DEV - 1):
            kr = pltpu.make_async_remote_copy(
                src_ref=kall.at[h], dst_ref=kall.at[h + 1],
                send_sem=ksend.at[h], recv_sem=krecv.at[h],
                device_id=(right,), device_id_type=pl.DeviceIdType.MESH)
            vr = pltpu.make_async_remote_copy(
                src_ref=vall.at[h], dst_ref=vall.at[h + 1],
                send_sem=vsend.at[h], recv_sem=vrecv.at[h],
                device_id=(right,), device_id_type=pl.DeviceIdType.MESH)
            kr.start()
            vr.start()
            kr.wait()
            vr.wait()

        for b in range(B):
            xb = x_ref[b].astype(jnp.bfloat16)
            qb = rope(jnp.dot(xb, wq, preferred_element_type=jnp.float32))
            ctx_heads = []
            for hh in range(HQ):
                qbh = qb[:, hh * DH:(hh + 1) * DH]
                s_blocks = []
                for j in range(N_DEV):
                    kbh = kall[j, b, :, hh * DH:(hh + 1) * DH]
                    s_blocks.append(lax.dot_general(
                        qbh, kbh, (((1,), (1,)), ((), ())),
                        preferred_element_type=jnp.float32))
                s = jnp.concatenate(s_blocks, axis=1) * 0.125
                m_ = jnp.max(s, axis=-1, keepdims=True)
                e = jnp.exp(s - m_)
                w = (e / jnp.sum(e, axis=-1, keepdims=True)).astype(jnp.bfloat16)
                ctx = None
                for j in range(N_DEV):
                    vbh = vall[j, b, :, hh * DH:(hh + 1) * DH]
                    part = jnp.dot(w[:, j * S_LOC:(j + 1) * S_LOC], vbh,
                                   preferred_element_type=jnp.float32)
                    ctx = part if ctx is None else ctx + part
                ctx_heads.append(ctx)
            ctxb = jnp.concatenate(ctx_heads, axis=1).astype(jnp.bfloat16)
            out_ref[b] = jnp.dot(ctxb, wo, preferred_element_type=jnp.float32)

    return pl.pallas_call(
        body,
        out_shape=jax.ShapeDtypeStruct((B, S_LOC, D_OUT), jnp.float32),
        in_specs=[pl.BlockSpec(memory_space=pltpu.VMEM)] * 5,
        out_specs=pl.BlockSpec(memory_space=pltpu.VMEM),
        scratch_shapes=[
            pltpu.VMEM((N_DEV, B, S_LOC, DQ), jnp.bfloat16),
            pltpu.VMEM((N_DEV, B, S_LOC, DQ), jnp.bfloat16),
            pltpu.SemaphoreType.DMA((N_DEV - 1,)),
            pltpu.SemaphoreType.DMA((N_DEV - 1,)),
            pltpu.SemaphoreType.DMA((N_DEV - 1,)),
            pltpu.SemaphoreType.DMA((N_DEV - 1,)),
        ],
        compiler_params=pltpu.CompilerParams(collective_id=0),
    )(x, Wq, Wk, Wv, Wo)


# baseline (device time: 28345 ns/iter reference)
import math

import jax
import jax.numpy as jnp
from jax import lax
from jax.experimental import pallas as pl
from jax.experimental.pallas import tpu as pltpu

N_DEV = 4
HQ, DH = 4, 64


def kernel(x, Wq, Wk, Wv, Wo):
    B, S_LOC, D = x.shape
    DQ = Wq.shape[1]
    D_OUT = Wo.shape[1]
    S_GLB = N_DEV * S_LOC

    def body(x_ref, wq_ref, wk_ref, wv_ref, wo_ref, out_ref,
             kall, vall, ksend, krecv, vsend, vrecv):
        my = lax.axis_index("i")
        right = lax.rem(my + 1, N_DEV)
        left = lax.rem(my + N_DEV - 1, N_DEV)

        barrier = pltpu.get_barrier_semaphore()
        for nbr in (left, right):
            pl.semaphore_signal(barrier, inc=1, device_id=(nbr,),
                                device_id_type=pl.DeviceIdType.MESH)
        pl.semaphore_wait(barrier, 2)

        row = lax.broadcasted_iota(jnp.int32, (S_LOC, DQ), 0).astype(jnp.float32)
        col = lax.broadcasted_iota(jnp.int32, (S_LOC, DQ), 1)
        dd = col % DH
        dpair = ((dd // 2) * 2).astype(jnp.float32)
        freq = jnp.exp(dpair * (-math.log(10000.0) / DH))
        pos = row + (my * S_LOC).astype(jnp.float32)
        ang = pos * freq
        cosv = jnp.cos(ang)
        sinv = jnp.sin(ang)

        r_i = lax.broadcasted_iota(jnp.int32, (DQ, DQ), 0)
        c_i = lax.broadcasted_iota(jnp.int32, (DQ, DQ), 1)
        M = jnp.where((r_i == c_i + 1) & (c_i % 2 == 0), -1.0,
                      jnp.where((r_i + 1 == c_i) & (c_i % 2 == 1), 1.0,
                                0.0)).astype(jnp.bfloat16)

        def rope(t):
            tr = jnp.dot(t.astype(jnp.bfloat16), M,
                         preferred_element_type=jnp.float32)
            return (t * cosv + tr * sinv).astype(jnp.bfloat16)

        wq = wq_ref[...].astype(jnp.bfloat16)
        wk = wk_ref[...].astype(jnp.bfloat16)
        wv = wv_ref[...].astype(jnp.bfloat16)
        wo = wo_ref[...].astype(jnp.bfloat16)

        for b in range(B):
            xb = x_ref[b].astype(jnp.bfloat16)
            kb = jnp.dot(xb, wk, preferred_element_type=jnp.float32)
            vb = jnp.dot(xb, wv, preferred_element_type=jnp.float32)
            kall[0, b] = rope(kb)
            vall[0, b] = vb.astype(jnp.bfloat16)

        for h in range(N_DEV - 1):
            kr = pltpu.make_async_remote_copy(
                src_ref=kall.at[h], dst_ref=kall.at[h + 1],
                send_sem=ksend.at[h], recv_sem=krecv.at[h],
                device_id=(right,), device_id_type=pl.DeviceIdType.MESH)
            vr = pltpu.make_async_remote_copy(
                src_ref=vall.at[h], dst_ref=vall.at[h + 1],
                send_sem=vsend.at[h], recv_sem=vrecv.at[h],
                device_id=(right,), device_id_type=pl.DeviceIdType.MESH)
            kr.start()
            vr.start()
            kr.wait()
            vr.wait()

        for b in range(B):
            xb = x_ref[b].astype(jnp.bfloat16)
            qb = rope(jnp.dot(xb, wq, preferred_element_type=jnp.float32))
            ctx_heads = []
            for hh in range(HQ):
                qbh = qb[:, hh * DH:(hh + 1) * DH]
                s_blocks = []
                for j in range(N_DEV):
                    kbh = kall[j, b, :, hh * DH:(hh + 1) * DH]
                    s_blocks.append(lax.dot_general(
                        qbh, kbh, (((1,), (1,)), ((), ())),
                        preferred_element_type=jnp.float32))
                s = jnp.concatenate(s_blocks, axis=1) * 0.125
                m_ = jnp.max(s, axis=-1, keepdims=True)
                e = jnp.exp(s - m_)
                w = (e / jnp.sum(e, axis=-1, keepdims=True)).astype(jnp.bfloat16)
                ctx = None
                for j in range(N_DEV):
                    vbh = vall[j, b, :, hh * DH:(hh + 1) * DH]
                    part = jnp.dot(w[:, j * S_LOC:(j + 1) * S_LOC], vbh,
                                   preferred_element_type=jnp.float32)
                    ctx = part if ctx is None else ctx + part
                ctx_heads.append(ctx)
            ctxb = jnp.concatenate(ctx_heads, axis=1).astype(jnp.bfloat16)
            out_ref[b] = jnp.dot(ctxb, wo, preferred_element_type=jnp.float32)

    return pl.pallas_call(
        body,
        out_shape=jax.ShapeDtypeStruct((B, S_LOC, D_OUT), jnp.float32),
        in_specs=[pl.BlockSpec(memory_space=pltpu.VMEM)] * 5,
        out_specs=pl.BlockSpec(memory_space=pltpu.VMEM),
        scratch_shapes=[
            pltpu.VMEM((N_DEV, B, S_LOC, DQ), jnp.bfloat16),
            pltpu.VMEM((N_DEV, B, S_LOC, DQ), jnp.bfloat16),
            pltpu.SemaphoreType.DMA((N_DEV - 1,)),
            pltpu.SemaphoreType.DMA((N_DEV - 1,)),
            pltpu.SemaphoreType.DMA((N_DEV - 1,)),
            pltpu.SemaphoreType.DMA((N_DEV - 1,)),
        ],
        compiler_params=pltpu.CompilerParams(collective_id=0),
    )(x, Wq, Wk, Wv, Wo)


# device time: 18988 ns/iter; 1.4928x vs baseline; 1.4928x over previous
import math

import jax
import jax.numpy as jnp
from jax import lax
from jax.experimental import pallas as pl
from jax.experimental.pallas import tpu as pltpu

N_DEV = 4
HQ, DH = 4, 64


def kernel(x, Wq, Wk, Wv, Wo):
    B, S_LOC, D = x.shape
    DQ = Wq.shape[1]
    D_OUT = Wo.shape[1]

    def body(x_ref, wq_ref, wk_ref, wv_ref, wo_ref, out_ref,
             kall, vall, ksend, krecv, vsend, vrecv):
        my = lax.axis_index("i")

        barrier = pltpu.get_barrier_semaphore()
        for o in range(1, N_DEV):
            peer = lax.rem(my + o, N_DEV)
            pl.semaphore_signal(barrier, inc=1, device_id=(peer,),
                                device_id_type=pl.DeviceIdType.MESH)
        pl.semaphore_wait(barrier, N_DEV - 1)

        row = lax.broadcasted_iota(jnp.int32, (S_LOC, DQ), 0).astype(jnp.float32)
        col = lax.broadcasted_iota(jnp.int32, (S_LOC, DQ), 1)
        dd = col % DH
        dpair = ((dd // 2) * 2).astype(jnp.float32)
        freq = jnp.exp(dpair * (-math.log(10000.0) / DH))
        pos = row + (my * S_LOC).astype(jnp.float32)
        ang = pos * freq
        cosv = jnp.cos(ang)
        sinv = jnp.sin(ang)

        r_i = lax.broadcasted_iota(jnp.int32, (DQ, DQ), 0)
        c_i = lax.broadcasted_iota(jnp.int32, (DQ, DQ), 1)
        M = jnp.where((r_i == c_i + 1) & (c_i % 2 == 0), -1.0,
                      jnp.where((r_i + 1 == c_i) & (c_i % 2 == 1), 1.0,
                                0.0)).astype(jnp.bfloat16)

        def rope(t):
            tr = jnp.dot(t.astype(jnp.bfloat16), M,
                         preferred_element_type=jnp.float32)
            return (t * cosv + tr * sinv).astype(jnp.bfloat16)

        wq = wq_ref[...].astype(jnp.bfloat16)
        wk = wk_ref[...].astype(jnp.bfloat16)
        wv = wv_ref[...].astype(jnp.bfloat16)
        wo = wo_ref[...].astype(jnp.bfloat16)

        for b in range(B):
            xb = x_ref[b].astype(jnp.bfloat16)
            kb = jnp.dot(xb, wk, preferred_element_type=jnp.float32)
            vb = jnp.dot(xb, wv, preferred_element_type=jnp.float32)
            kall[0, b] = rope(kb)
            vall[0, b] = vb.astype(jnp.bfloat16)

        krdmas, vrdmas = [], []
        for o in range(1, N_DEV):
            peer = lax.rem(my + o, N_DEV)
            slot = N_DEV - o
            kr = pltpu.make_async_remote_copy(
                src_ref=kall.at[0], dst_ref=kall.at[slot],
                send_sem=ksend.at[o - 1], recv_sem=krecv.at[slot],
                device_id=(peer,), device_id_type=pl.DeviceIdType.MESH)
            vr = pltpu.make_async_remote_copy(
                src_ref=vall.at[0], dst_ref=vall.at[slot],
                send_sem=vsend.at[o - 1], recv_sem=vrecv.at[slot],
                device_id=(peer,), device_id_type=pl.DeviceIdType.MESH)
            kr.start()
            vr.start()
            krdmas.append(kr)
            vrdmas.append(vr)

        qs = []
        for b in range(B):
            xb = x_ref[b].astype(jnp.bfloat16)
            qs.append(rope(jnp.dot(xb, wq, preferred_element_type=jnp.float32)))

        s_blocks = [[[] for _ in range(HQ)] for _ in range(B)]

        def add_slot_scores(j):
            for b in range(B):
                for hh in range(HQ):
                    qbh = qs[b][:, hh * DH:(hh + 1) * DH]
                    kbh = kall[j, b, :, hh * DH:(hh + 1) * DH]
                    s_blocks[b][hh].append(lax.dot_general(
                        qbh, kbh, (((1,), (1,)), ((), ())),
                        preferred_element_type=jnp.float32))

        add_slot_scores(0)
        for j in range(1, N_DEV):
            krdmas[N_DEV - 1 - j].wait_recv()
            add_slot_scores(j)

        ws = [[None] * HQ for _ in range(B)]
        for b in range(B):
            for hh in range(HQ):
                s = jnp.concatenate(s_blocks[b][hh], axis=1) * 0.125
                m_ = jnp.max(s, axis=-1, keepdims=True)
                e = jnp.exp(s - m_)
                ws[b][hh] = (e / jnp.sum(e, axis=-1, keepdims=True)
                             ).astype(jnp.bfloat16)

        for vr in vrdmas:
            vr.wait_recv()
        for b in range(B):
            ctx_heads = []
            for hh in range(HQ):
                ctx = None
                for j in range(N_DEV):
                    vbh = vall[j, b, :, hh * DH:(hh + 1) * DH]
                    part = jnp.dot(ws[b][hh][:, j * S_LOC:(j + 1) * S_LOC],
                                   vbh, preferred_element_type=jnp.float32)
                    ctx = part if ctx is None else ctx + part
                ctx_heads.append(ctx)
            ctxb = jnp.concatenate(ctx_heads, axis=1).astype(jnp.bfloat16)
            out_ref[b] = jnp.dot(ctxb, wo, preferred_element_type=jnp.float32)

        for kr in krdmas:
            kr.wait_send()
        for vr in vrdmas:
            vr.wait_send()

    return pl.pallas_call(
        body,
        out_shape=jax.ShapeDtypeStruct((B, S_LOC, D_OUT), jnp.float32),
        in_specs=[pl.BlockSpec(memory_space=pltpu.VMEM)] * 5,
        out_specs=pl.BlockSpec(memory_space=pltpu.VMEM),
        scratch_shapes=[
            pltpu.VMEM((N_DEV, B, S_LOC, DQ), jnp.bfloat16),
            pltpu.VMEM((N_DEV, B, S_LOC, DQ), jnp.bfloat16),
            pltpu.SemaphoreType.DMA((N_DEV - 1,)),
            pltpu.SemaphoreType.DMA((N_DEV,)),
            pltpu.SemaphoreType.DMA((N_DEV - 1,)),
            pltpu.SemaphoreType.DMA((N_DEV,)),
        ],
        compiler_params=pltpu.CompilerParams(collective_id=0),
    )(x, Wq, Wk, Wv, Wo)


# device time: 18889 ns/iter; 1.5006x vs baseline; 1.0052x over previous
import math

import jax
import jax.numpy as jnp
from jax import lax
from jax.experimental import pallas as pl
from jax.experimental.pallas import tpu as pltpu

N_DEV = 4
HQ, DH = 4, 64


def kernel(x, Wq, Wk, Wv, Wo):
    B, S_LOC, D = x.shape
    DQ = Wq.shape[1]
    D_OUT = Wo.shape[1]

    def body(x_ref, wq_ref, wk_ref, wv_ref, wo_ref, out_ref,
             kall, vall, ksend, krecv, vsend, vrecv):
        my = lax.axis_index("i")

        barrier = pltpu.get_barrier_semaphore()
        for o in range(1, N_DEV):
            peer = lax.rem(my + o, N_DEV)
            pl.semaphore_signal(barrier, inc=1, device_id=(peer,),
                                device_id_type=pl.DeviceIdType.MESH)
        pl.semaphore_wait(barrier, N_DEV - 1)

        row = lax.broadcasted_iota(jnp.int32, (S_LOC, DQ), 0).astype(jnp.float32)
        col = lax.broadcasted_iota(jnp.int32, (S_LOC, DQ), 1)
        dd = col % DH
        dpair = ((dd // 2) * 2).astype(jnp.float32)
        freq = jnp.exp(dpair * (-math.log(10000.0) / DH))
        pos = row + (my * S_LOC).astype(jnp.float32)
        ang = pos * freq
        cosv = jnp.cos(ang)
        sinv = jnp.sin(ang)

        r_i = lax.broadcasted_iota(jnp.int32, (DQ, DQ), 0)
        c_i = lax.broadcasted_iota(jnp.int32, (DQ, DQ), 1)
        M = jnp.where((r_i == c_i + 1) & (c_i % 2 == 0), -1.0,
                      jnp.where((r_i + 1 == c_i) & (c_i % 2 == 1), 1.0,
                                0.0)).astype(jnp.bfloat16)

        def rope(t):
            tr = jnp.dot(t.astype(jnp.bfloat16), M,
                         preferred_element_type=jnp.float32)
            return (t * cosv + tr * sinv).astype(jnp.bfloat16)

        wq = wq_ref[...].astype(jnp.bfloat16)
        wk = wk_ref[...].astype(jnp.bfloat16)
        wv = wv_ref[...].astype(jnp.bfloat16)
        wo = wo_ref[...].astype(jnp.bfloat16)

        for b in range(B):
            xb = x_ref[b].astype(jnp.bfloat16)
            kb = jnp.dot(xb, wk, preferred_element_type=jnp.float32)
            kall[0, b] = rope(kb)
        krdmas = {}
        for o in range(1, N_DEV):
            peer = lax.rem(my + o, N_DEV)
            slot = N_DEV - o
            kr = pltpu.make_async_remote_copy(
                src_ref=kall.at[0], dst_ref=kall.at[slot],
                send_sem=ksend.at[o - 1], recv_sem=krecv.at[slot],
                device_id=(peer,), device_id_type=pl.DeviceIdType.MESH)
            kr.start()
            krdmas[slot] = kr

        for b in range(B):
            xb = x_ref[b].astype(jnp.bfloat16)
            vb = jnp.dot(xb, wv, preferred_element_type=jnp.float32)
            vall[0, b] = vb.astype(jnp.bfloat16)
        vrdmas = {}
        for o in range(1, N_DEV):
            peer = lax.rem(my + o, N_DEV)
            slot = N_DEV - o
            vr = pltpu.make_async_remote_copy(
                src_ref=vall.at[0], dst_ref=vall.at[slot],
                send_sem=vsend.at[o - 1], recv_sem=vrecv.at[slot],
                device_id=(peer,), device_id_type=pl.DeviceIdType.MESH)
            vr.start()
            vrdmas[slot] = vr

        qs = []
        for b in range(B):
            xb = x_ref[b].astype(jnp.bfloat16)
            qs.append(rope(jnp.dot(xb, wq, preferred_element_type=jnp.float32)))

        SLOT_ORDER = (0, 1, 3, 2)

        s_blocks = [[[] for _ in range(HQ)] for _ in range(B)]

        def add_slot_scores(j):
            for b in range(B):
                for hh in range(HQ):
                    qbh = qs[b][:, hh * DH:(hh + 1) * DH]
                    kbh = kall[j, b, :, hh * DH:(hh + 1) * DH]
                    s_blocks[b][hh].append(lax.dot_general(
                        qbh, kbh, (((1,), (1,)), ((), ())),
                        preferred_element_type=jnp.float32))

        add_slot_scores(0)
        for j in SLOT_ORDER[1:]:
            krdmas[j].wait_recv()
            add_slot_scores(j)

        ws = [[None] * HQ for _ in range(B)]
        for b in range(B):
            for hh in range(HQ):
                s = jnp.concatenate(s_blocks[b][hh], axis=1) * 0.125
                m_ = jnp.max(s, axis=-1, keepdims=True)
                e = jnp.exp(s - m_)
                ws[b][hh] = (e / jnp.sum(e, axis=-1, keepdims=True)
                             ).astype(jnp.bfloat16)

        for j in SLOT_ORDER[1:]:
            vrdmas[j].wait_recv()
        for b in range(B):
            ctx_heads = []
            for hh in range(HQ):
                ctx = None
                for idx, j in enumerate(SLOT_ORDER):
                    vbh = vall[j, b, :, hh * DH:(hh + 1) * DH]
                    part = jnp.dot(ws[b][hh][:, idx * S_LOC:(idx + 1) * S_LOC],
                                   vbh, preferred_element_type=jnp.float32)
                    ctx = part if ctx is None else ctx + part
                ctx_heads.append(ctx)
            ctxb = jnp.concatenate(ctx_heads, axis=1).astype(jnp.bfloat16)
            out_ref[b] = jnp.dot(ctxb, wo, preferred_element_type=jnp.float32)

        for kr in krdmas.values():
            kr.wait_send()
        for vr in vrdmas.values():
            vr.wait_send()

    return pl.pallas_call(
        body,
        out_shape=jax.ShapeDtypeStruct((B, S_LOC, D_OUT), jnp.float32),
        in_specs=[pl.BlockSpec(memory_space=pltpu.VMEM)] * 5,
        out_specs=pl.BlockSpec(memory_space=pltpu.VMEM),
        scratch_shapes=[
            pltpu.VMEM((N_DEV, B, S_LOC, DQ), jnp.bfloat16),
            pltpu.VMEM((N_DEV, B, S_LOC, DQ), jnp.bfloat16),
            pltpu.SemaphoreType.DMA((N_DEV - 1,)),
            pltpu.SemaphoreType.DMA((N_DEV,)),
            pltpu.SemaphoreType.DMA((N_DEV - 1,)),
            pltpu.SemaphoreType.DMA((N_DEV,)),
        ],
        compiler_params=pltpu.CompilerParams(collective_id=0),
    )(x, Wq, Wk, Wv, Wo)
